# baseline (speedup 1.0000x reference)
import functools

import jax
import jax.numpy as jnp
import numpy as np
from jax.experimental import pallas as pl
from jax.experimental.pallas import tpu as pltpu

_BLOCK_SIZE = 32
_BLOCK_COUNTS = 4


def _hstu_bsa_kernel(q_ref, k_ref, v_ref, gc_ref, gs_ref, maskbd_ref,
                     cbt_ref, poolbd_ref, causal_ref, gmat_ref, out_ref,
                     *, bs, ssel, H, D):
    f32 = jnp.float32
    bf = jnp.bfloat16
    hi = jax.lax.Precision.HIGHEST
    L = q_ref.shape[0]
    nblk = L // bs
    HD = H * D
    scale = D ** (-0.5)

    q_all = q_ref[...]
    k_all = k_ref[...]
    v_all = v_ref[...]
    gc8 = gc_ref[...]
    gs8 = gs_ref[...]
    mask_bd = maskbd_ref[...]
    cbt = cbt_ref[...]
    pool_bd = poolbd_ref[...]
    causal2 = causal_ref[...]
    gmat = gmat_ref[...]

    k_cmp = jnp.sum(k_all.reshape(nblk, bs, HD), axis=1) * (1.0 / bs)
    v_cmp = jnp.sum(v_all.reshape(nblk, bs, HD), axis=1,
                    dtype=f32) * (1.0 / bs)

    kcb = (jnp.concatenate([k_cmp] * H, axis=0) * mask_bd).astype(bf)
    vcb = (jnp.concatenate([v_cmp] * H, axis=0) * mask_bd).astype(bf)

    s_t = jax.lax.dot_general(
        kcb, q_all, (((1,), (1,)), ((), ())),
        preferred_element_type=f32) * scale
    s_sel = jnp.where(cbt > 0.5, s_t, -jnp.inf)

    s3 = s_sel.reshape(H, nblk, L)
    blk3 = jax.lax.broadcasted_iota(jnp.int32, (H, nblk, L), 1)
    rank = jnp.zeros((H, nblk, L), dtype=f32)
    for j in range(nblk):
        sj = jnp.broadcast_to(s3[:, j:j + 1, :], (H, nblk, L))
        beats = (sj > s3) | ((sj == s3) & (j < blk3))
        rank = rank + beats.astype(f32)
    sel_t = (rank < ssel).astype(f32).reshape(H * nblk, L)

    sel_q = jnp.transpose(sel_t).astype(bf)
    mask_all = jnp.dot(sel_q, pool_bd, preferred_element_type=f32)

    p_cmp_t = jnp.where(cbt > 0.5, s_t * jax.nn.sigmoid(s_t), 0.0)
    p_cmp = jnp.transpose(p_cmp_t).astype(bf)
    out_cmp = jnp.dot(p_cmp, vcb, preferred_element_type=f32)

    lane = jax.lax.broadcasted_iota(jnp.int32, (L, 2 * D), 1)
    m0 = (lane < D).astype(f32)
    m1 = 1.0 - m0
    m0b = m0.astype(bf)
    m1b = m1.astype(bf)
    outs = []
    for g in range(H // 2):
        sl = slice(g * 2 * D, (g + 1) * 2 * D)
        q2 = q_all[:, sl]
        k2 = k_all[:, sl]
        v2 = v_all[:, sl]
        k_st = jnp.concatenate([k2 * m0, k2 * m1], axis=0)
        v_st = jnp.concatenate([v2 * m0b, v2 * m1b], axis=0)
        s2 = jax.lax.dot_general(
            q2, k_st.astype(bf), (((1,), (1,)), ((), ())),
            preferred_element_type=f32) * scale
        p2 = (s2 * jax.nn.sigmoid(s2) * causal2
              * mask_all[:, g * 2 * L:(g + 1) * 2 * L])
        outs.append(jnp.dot(p2.astype(bf), v_st,
                            preferred_element_type=f32))
    out_slc = jnp.concatenate(outs, axis=1)

    gidx = jax.lax.broadcasted_iota(jnp.int32, (L, HD), 1) // D
    gc_b = jnp.take_along_axis(gc8, gidx, axis=1)
    gs_b = jnp.take_along_axis(gs8, gidx, axis=1)
    out_ref[...] = out_cmp * gc_b + out_slc * gs_b


def kernel(q, k, v, g_cmp, g_slc, x_offsets):
    T, H, D = q.shape
    Bn = x_offsets.shape[0] - 1
    L = T // Bn
    bs = _BLOCK_SIZE
    nblk = L // bs
    HD = H * D

    qf = q.reshape(T, HD).astype(jnp.bfloat16)
    kf = k.reshape(T, HD)
    vf = v.reshape(T, HD).astype(jnp.bfloat16)
    gcf = g_cmp.reshape(T, H)
    gsf = g_slc.reshape(T, H)

    r_hb = np.arange(H * nblk)
    c_hd = np.arange(HD)
    mask_bd = (r_hb[:, None] // nblk == c_hd[None, :] // D).astype(np.float32)
    qpos = np.arange(L)
    cbt = (r_hb[:, None] % nblk <= qpos[None, :] // bs).astype(np.float32)
    c_ht = np.arange(H * L)
    pool_bd = jnp.asarray(
        (r_hb[:, None] // nblk == c_ht[None, :] // L)
        & (r_hb[:, None] % nblk == (c_ht[None, :] % L) // bs),
        dtype=jnp.bfloat16)
    c_2t = np.arange(2 * L)
    causal2 = (c_2t[None, :] % L <= qpos[:, None]).astype(np.float32)
    gmat = (np.arange(H)[:, None] == c_hd[None, :] // D).astype(np.float32)

    def const_spec(shape):
        return pl.BlockSpec(shape, lambda b: (0,) * len(shape))

    body = functools.partial(_hstu_bsa_kernel, bs=bs,
                             ssel=min(_BLOCK_COUNTS, nblk), H=H, D=D)
    out = pl.pallas_call(
        body,
        grid=(Bn,),
        in_specs=[
            pl.BlockSpec((L, HD), lambda b: (b, 0)),
            pl.BlockSpec((L, HD), lambda b: (b, 0)),
            pl.BlockSpec((L, HD), lambda b: (b, 0)),
            pl.BlockSpec((L, H), lambda b: (b, 0)),
            pl.BlockSpec((L, H), lambda b: (b, 0)),
            const_spec((H * nblk, HD)),
            const_spec((H * nblk, L)),
            const_spec((H * nblk, H * L)),
            const_spec((L, 2 * L)),
            const_spec((H, HD)),
        ],
        out_specs=pl.BlockSpec((L, HD), lambda b: (b, 0)),
        out_shape=jax.ShapeDtypeStruct((T, HD), jnp.float32),
        compiler_params=pltpu.CompilerParams(
            dimension_semantics=("parallel",)),
    )(qf, kf, vf, gcf, gsf, mask_bd, cbt, pool_bd, causal2, gmat)

    return out.reshape(T, H, D)


# speedup vs baseline: 1.0336x; 1.0336x over previous
import functools

import jax
import jax.numpy as jnp
import numpy as np
from jax.experimental import pallas as pl
from jax.experimental.pallas import tpu as pltpu

_BLOCK_SIZE = 32
_BLOCK_COUNTS = 4


def _hstu_bsa_kernel(q_ref, k_ref, v_ref, gc_ref, gs_ref, maskbd_ref,
                     cbt_ref, poolbd_ref, causal_ref, out_ref,
                     *, bs, ssel, H, D):
    f32 = jnp.float32
    bf = jnp.bfloat16
    L = q_ref.shape[0]
    nblk = L // bs
    HD = H * D
    scale = D ** (-0.5)

    q_all = q_ref[...]
    k_all = k_ref[...]
    v_all = v_ref[...]
    gc8 = gc_ref[...]
    gs8 = gs_ref[...]
    mask_bd = maskbd_ref[...]
    cbt = cbt_ref[...]
    pool_bd = poolbd_ref[...]
    causal2 = causal_ref[...]

    k_cmp = jnp.sum(k_all.reshape(nblk, bs, HD), axis=1) * (1.0 / bs)
    v_cmp = jnp.sum(v_all.reshape(nblk, bs, HD), axis=1) * (1.0 / bs)

    kcb = (jnp.concatenate([k_cmp] * H, axis=0) * mask_bd).astype(bf)
    vcb = (jnp.concatenate([v_cmp] * H, axis=0) * mask_bd).astype(bf)

    s_t = jax.lax.dot_general(
        kcb, q_all, (((1,), (1,)), ((), ())),
        preferred_element_type=f32) * scale
    s_sel = jnp.where(cbt > 0.5, s_t, -jnp.inf)

    s3 = s_sel.reshape(H, nblk, L)
    blk3 = jax.lax.broadcasted_iota(jnp.int32, (H, nblk, L), 1)
    rank = jnp.zeros((H, nblk, L), dtype=f32)
    for j in range(nblk):
        sj = jnp.broadcast_to(s3[:, j:j + 1, :], (H, nblk, L))
        beats = (sj > s3) | ((sj == s3) & (j < blk3))
        rank = rank + beats.astype(f32)
    sel_t = (rank < ssel).astype(f32).reshape(H * nblk, L)

    sel_q = jnp.transpose(sel_t).astype(bf)
    mask_all = jnp.dot(sel_q, pool_bd, preferred_element_type=f32)

    p_cmp_t = jnp.where(cbt > 0.5, s_t * jax.nn.sigmoid(s_t), 0.0)
    p_cmp = jnp.transpose(p_cmp_t).astype(bf)
    out_cmp = jnp.dot(p_cmp, vcb, preferred_element_type=f32)

    lane = jax.lax.broadcasted_iota(jnp.int32, (L, 2 * D), 1)
    m0 = (lane < D).astype(f32)
    m1 = 1.0 - m0
    outs = []
    for g in range(H // 2):
        sl = slice(g * 2 * D, (g + 1) * 2 * D)
        q2 = q_all[:, sl]
        k2 = k_all[:, sl]
        v2 = v_all[:, sl]
        k_st = jnp.concatenate([k2 * m0, k2 * m1], axis=0)
        v_st = jnp.concatenate([v2 * m0, v2 * m1], axis=0)
        s2 = jax.lax.dot_general(
            q2, k_st.astype(bf), (((1,), (1,)), ((), ())),
            preferred_element_type=f32) * scale
        p2 = (s2 * jax.nn.sigmoid(s2) * causal2
              * mask_all[:, g * 2 * L:(g + 1) * 2 * L])
        outs.append(jnp.dot(p2.astype(bf), v_st.astype(bf),
                            preferred_element_type=f32))
    out_slc = jnp.concatenate(outs, axis=1)

    gidx = jax.lax.broadcasted_iota(jnp.int32, (L, HD), 1) // D
    gc_b = jnp.take_along_axis(gc8, gidx, axis=1)
    gs_b = jnp.take_along_axis(gs8, gidx, axis=1)
    out_ref[...] = out_cmp * gc_b + out_slc * gs_b


def kernel(q, k, v, g_cmp, g_slc, x_offsets):
    T, H, D = q.shape
    Bn = x_offsets.shape[0] - 1
    L = T // Bn
    bs = _BLOCK_SIZE
    nblk = L // bs
    HD = H * D

    qf = q.reshape(T, HD).astype(jnp.bfloat16)
    kf = k.reshape(T, HD)
    vf = v.reshape(T, HD)
    gcf = g_cmp.reshape(T, H)
    gsf = g_slc.reshape(T, H)

    r_hb = np.arange(H * nblk)
    c_hd = np.arange(HD)
    mask_bd = (r_hb[:, None] // nblk == c_hd[None, :] // D).astype(np.float32)
    qpos = np.arange(L)
    cbt = (r_hb[:, None] % nblk <= qpos[None, :] // bs).astype(np.float32)
    c_ht = np.arange(H * L)
    pool_bd = jnp.asarray(
        (r_hb[:, None] // nblk == c_ht[None, :] // L)
        & (r_hb[:, None] % nblk == (c_ht[None, :] % L) // bs),
        dtype=jnp.bfloat16)
    c_2t = np.arange(2 * L)
    causal2 = (c_2t[None, :] % L <= qpos[:, None]).astype(np.float32)

    def const_spec(shape):
        return pl.BlockSpec(shape, lambda b: (0,) * len(shape))

    body = functools.partial(_hstu_bsa_kernel, bs=bs,
                             ssel=min(_BLOCK_COUNTS, nblk), H=H, D=D)
    out = pl.pallas_call(
        body,
        grid=(Bn,),
        in_specs=[
            pl.BlockSpec((L, HD), lambda b: (b, 0)),
            pl.BlockSpec((L, HD), lambda b: (b, 0)),
            pl.BlockSpec((L, HD), lambda b: (b, 0)),
            pl.BlockSpec((L, H), lambda b: (b, 0)),
            pl.BlockSpec((L, H), lambda b: (b, 0)),
            const_spec((H * nblk, HD)),
            const_spec((H * nblk, L)),
            const_spec((H * nblk, H * L)),
            const_spec((L, 2 * L)),
        ],
        out_specs=pl.BlockSpec((L, HD), lambda b: (b, 0)),
        out_shape=jax.ShapeDtypeStruct((T, HD), jnp.float32),
        compiler_params=pltpu.CompilerParams(
            dimension_semantics=("parallel",)),
    )(qf, kf, vf, gcf, gsf, mask_bd, cbt, pool_bd, causal2)

    return out.reshape(T, H, D)


# speedup vs baseline: 1.0719x; 1.0371x over previous
import functools

import jax
import jax.numpy as jnp
import numpy as np
from jax.experimental import pallas as pl
from jax.experimental.pallas import tpu as pltpu

_BLOCK_SIZE = 32
_BLOCK_COUNTS = 4


def _hstu_bsa_kernel(q_ref, k_ref, v_ref, gc_ref, gs_ref, maskbd_ref,
                     cbt_ref, poolbd_ref, causal_ref, out_ref,
                     *, bs, ssel, H, D):
    f32 = jnp.float32
    bf = jnp.bfloat16
    L = q_ref.shape[0]
    nblk = L // bs
    HD = H * D
    scale = D ** (-0.5)

    q_all = q_ref[...]
    k_all = k_ref[...]
    v_all = v_ref[...]
    gc8 = gc_ref[...]
    gs8 = gs_ref[...]
    mask_bd = maskbd_ref[...]
    cbt = cbt_ref[...]
    pool_bd = poolbd_ref[...]
    causal2 = causal_ref[...]

    k_cmp = jnp.sum(k_all.reshape(nblk, bs, HD), axis=1) * (1.0 / bs)
    v_cmp = jnp.sum(v_all.reshape(nblk, bs, HD), axis=1) * (1.0 / bs)

    kcb = (jnp.concatenate([k_cmp] * H, axis=0) * mask_bd).astype(bf)
    vcb = (jnp.concatenate([v_cmp] * H, axis=0) * mask_bd).astype(bf)

    s_t = jax.lax.dot_general(
        kcb, q_all, (((1,), (1,)), ((), ())),
        preferred_element_type=f32) * scale
    s_sel = jnp.where(cbt > 0.5, s_t, -jnp.inf)

    s3 = s_sel.reshape(H, nblk, L)
    blk3 = jax.lax.broadcasted_iota(jnp.int32, (H, nblk, L), 1)
    rank = jnp.zeros((H, nblk, L), dtype=f32)
    for j in range(nblk):
        sj = jnp.broadcast_to(s3[:, j:j + 1, :], (H, nblk, L))
        beats = (sj > s3) | ((sj == s3) & (j < blk3))
        rank = rank + beats.astype(f32)
    sel_t = (rank < ssel).astype(f32).reshape(H * nblk, L)

    sel_q = jnp.transpose(sel_t).astype(bf)
    mask_all = jnp.dot(sel_q, pool_bd, preferred_element_type=f32)

    p_cmp_t = jnp.where(cbt > 0.5, s_t * jax.nn.sigmoid(s_t), 0.0)
    p_cmp = jnp.transpose(p_cmp_t).astype(bf)
    out_cmp = jnp.dot(p_cmp, vcb, preferred_element_type=f32)

    L2 = L // 2
    lane = jax.lax.broadcasted_iota(jnp.int32, (L, 2 * D), 1)
    m0 = (lane < D).astype(f32)
    m1 = 1.0 - m0
    ca = jnp.concatenate([causal2[:L2, :L2]] * 2, axis=1)
    cb = causal2[L2:, :]
    outs = []
    for g in range(H // 2):
        sl = slice(g * 2 * D, (g + 1) * 2 * D)
        q2 = q_all[:, sl]
        k2 = k_all[:, sl]
        v2 = v_all[:, sl]
        ka0 = (k2[:L2] * m0[:L2]).astype(bf)
        ka1 = (k2[L2:] * m0[L2:]).astype(bf)
        kb0 = (k2[:L2] * m1[:L2]).astype(bf)
        kb1 = (k2[L2:] * m1[L2:]).astype(bf)
        va0 = (v2[:L2] * m0[:L2]).astype(bf)
        va1 = (v2[L2:] * m0[L2:]).astype(bf)
        vb0 = (v2[:L2] * m1[:L2]).astype(bf)
        vb1 = (v2[L2:] * m1[L2:]).astype(bf)
        k_st_a = jnp.concatenate([ka0, kb0], axis=0)
        k_st_b = jnp.concatenate([ka0, ka1, kb0, kb1], axis=0)
        v_st_a = jnp.concatenate([va0, vb0], axis=0)
        v_st_b = jnp.concatenate([va0, va1, vb0, vb1], axis=0)
        mask2 = mask_all[:, g * 2 * L:(g + 1) * 2 * L]
        mask2a = jnp.concatenate(
            [mask2[:L2, :L2], mask2[:L2, L:L + L2]], axis=1)
        mask2b = mask2[L2:, :]
        s2a = jax.lax.dot_general(
            q2[:L2], k_st_a, (((1,), (1,)), ((), ())),
            preferred_element_type=f32) * scale
        s2b = jax.lax.dot_general(
            q2[L2:], k_st_b, (((1,), (1,)), ((), ())),
            preferred_element_type=f32) * scale
        p2a = s2a * jax.nn.sigmoid(s2a) * ca * mask2a
        p2b = s2b * jax.nn.sigmoid(s2b) * cb * mask2b
        o2a = jnp.dot(p2a.astype(bf), v_st_a,
                      preferred_element_type=f32)
        o2b = jnp.dot(p2b.astype(bf), v_st_b,
                      preferred_element_type=f32)
        outs.append(jnp.concatenate([o2a, o2b], axis=0))
    out_slc = jnp.concatenate(outs, axis=1)

    gidx = jax.lax.broadcasted_iota(jnp.int32, (L, HD), 1) // D
    gc_b = jnp.take_along_axis(gc8, gidx, axis=1)
    gs_b = jnp.take_along_axis(gs8, gidx, axis=1)
    out_ref[...] = out_cmp * gc_b + out_slc * gs_b


def kernel(q, k, v, g_cmp, g_slc, x_offsets):
    T, H, D = q.shape
    Bn = x_offsets.shape[0] - 1
    L = T // Bn
    bs = _BLOCK_SIZE
    nblk = L // bs
    HD = H * D

    qf = q.reshape(T, HD).astype(jnp.bfloat16)
    kf = k.reshape(T, HD)
    vf = v.reshape(T, HD)
    gcf = g_cmp.reshape(T, H)
    gsf = g_slc.reshape(T, H)

    r_hb = np.arange(H * nblk)
    c_hd = np.arange(HD)
    mask_bd = (r_hb[:, None] // nblk == c_hd[None, :] // D).astype(np.float32)
    qpos = np.arange(L)
    cbt = (r_hb[:, None] % nblk <= qpos[None, :] // bs).astype(np.float32)
    c_ht = np.arange(H * L)
    pool_bd = jnp.asarray(
        (r_hb[:, None] // nblk == c_ht[None, :] // L)
        & (r_hb[:, None] % nblk == (c_ht[None, :] % L) // bs),
        dtype=jnp.bfloat16)
    c_2t = np.arange(2 * L)
    causal2 = (c_2t[None, :] % L <= qpos[:, None]).astype(np.float32)

    def const_spec(shape):
        return pl.BlockSpec(shape, lambda b: (0,) * len(shape))

    body = functools.partial(_hstu_bsa_kernel, bs=bs,
                             ssel=min(_BLOCK_COUNTS, nblk), H=H, D=D)
    out = pl.pallas_call(
        body,
        grid=(Bn,),
        in_specs=[
            pl.BlockSpec((L, HD), lambda b: (b, 0)),
            pl.BlockSpec((L, HD), lambda b: (b, 0)),
            pl.BlockSpec((L, HD), lambda b: (b, 0)),
            pl.BlockSpec((L, H), lambda b: (b, 0)),
            pl.BlockSpec((L, H), lambda b: (b, 0)),
            const_spec((H * nblk, HD)),
            const_spec((H * nblk, L)),
            const_spec((H * nblk, H * L)),
            const_spec((L, 2 * L)),
        ],
        out_specs=pl.BlockSpec((L, HD), lambda b: (b, 0)),
        out_shape=jax.ShapeDtypeStruct((T, HD), jnp.float32),
        compiler_params=pltpu.CompilerParams(
            dimension_semantics=("parallel",)),
    )(qf, kf, vf, gcf, gsf, mask_bd, cbt, pool_bd, causal2)

    return out.reshape(T, H, D)
